# baseline (speedup 1.0000x reference)
import functools

import jax
import jax.numpy as jnp
from jax import lax
from jax.experimental import pallas as pl
from jax.experimental.pallas import tpu as pltpu
from jax.experimental.pallas import tpu_sc as plsc

D = 128
SEQ = 200
HALFSEQ = 100
LANES = 16


@functools.lru_cache(maxsize=None)
def _make_emb_kernel(batch: int):
    info = plsc.get_sparse_core_info()
    nc, ns = info.num_cores, info.num_subcores
    nw = nc * ns
    seq_per_w = batch // nw
    rows_per_w = seq_per_w * SEQ
    n_u = (seq_per_w - 2) // 3
    n_tail = seq_per_w - 2 - 3 * n_u
    assert n_tail == 0 and seq_per_w >= 5

    mesh = plsc.VectorSubcoreMesh(core_axis_name="c", subcore_axis_name="s")

    @functools.partial(
        pl.kernel,
        out_type=jax.ShapeDtypeStruct((batch * SEQ, D), jnp.float32),
        mesh=mesh,
        scratch_types=[
            pltpu.VMEM((3, 2, HALFSEQ), jnp.int32),
            pltpu.VMEM((SEQ, D), jnp.float32),
            pltpu.VMEM((3, SEQ, D), jnp.float32),
            pltpu.SemaphoreType.DMA,
            pltpu.SemaphoreType.DMA,
            pltpu.SemaphoreType.DMA,
            pltpu.SemaphoreType.DMA,
            pltpu.SemaphoreType.DMA,
            pltpu.SemaphoreType.DMA,
            pltpu.SemaphoreType.DMA,
            pltpu.SemaphoreType.DMA,
            pltpu.SemaphoreType.DMA,
        ],
    )
    def emb(ids_hbm, pos_hbm, table_hbm, out_hbm, idx_v, pos_v, bufs,
            g0, g1, g2, s0, s1, s2, x0, x1, x2):
        g = (g0, g1, g2)
        st = (s0, s1, s2)
        ix = (x0, x1, x2)
        wid = lax.axis_index("s") * nc + lax.axis_index("c")
        pltpu.sync_copy(pos_hbm, pos_v)
        seq_w = wid * seq_per_w
        out_w = wid * rows_per_w

        def issue_idx(half, s):
            pltpu.async_copy(ids_hbm.at[seq_w + s], idx_v.at[half], ix[half])

        def wait_idx(half):
            pltpu.make_async_copy(
                ids_hbm.at[0], idx_v.at[half], ix[half]).wait()

        def prefill(half, _unused=None):
            @plsc.parallel_loop(0, SEQ, unroll=8)
            def row_body(r):
                for k in range(D // LANES):
                    sl = pl.ds(k * LANES, LANES)
                    bufs[half, r, sl] = pos_v[r, sl]

        def issue_gathers(half, s):
            del s
            for j in range(2):
                pltpu.async_copy(
                    table_hbm.at[idx_v.at[half, j]],
                    bufs.at[half, pl.ds(j * HALFSEQ, HALFSEQ)],
                    g[half], add=True)

        def wait_gathers(half):
            for j in range(2):
                pltpu.make_async_copy(
                    table_hbm.at[idx_v.at[0, 0]],
                    bufs.at[half, pl.ds(j * HALFSEQ, HALFSEQ)],
                    g[half]).wait()

        def issue_store(half, s):
            pltpu.async_copy(
                bufs.at[half], out_hbm.at[pl.ds(out_w + s * SEQ, SEQ)],
                st[half])

        def wait_store(half):
            pltpu.make_async_copy(
                bufs.at[half], out_hbm.at[pl.ds(0, SEQ)], st[half]).wait()

        for b in (0, 1, 2):
            issue_idx(b, b)
        wait_idx(0)
        prefill(0)
        issue_gathers(0, 0)
        wait_idx(1)
        prefill(1)
        issue_gathers(1, 1)

        def u_body(u, carry):
            for h in (0, 1, 2):
                nn = (h + 2) % 3
                s = 3 * u + h
                wait_gathers(h)
                if h == 2:
                    @pl.when(u < n_u - 1)
                    def _():
                        issue_idx(h, s + 3)
                else:
                    issue_idx(h, s + 3)
                issue_store(h, s)
                if h == 0:
                    @pl.when(u > 0)
                    def _():
                        wait_store(nn)
                else:
                    wait_store(nn)
                prefill(nn)
                wait_idx(nn)
                issue_gathers(nn, s + 2)
            return carry

        lax.fori_loop(0, n_u, u_body, 0)
        for h, s in ((0, seq_per_w - 2), (1, seq_per_w - 1)):
            wait_gathers(h)
            issue_store(h, s)
            wait_store((h + 2) % 3)
        wait_store(1)

    return emb


def kernel(input_ids, speaker_ids, charactor_embeddings, position_table):
    del speaker_ids
    batch, seq = input_ids.shape
    _, d = charactor_embeddings.shape
    ids3d = input_ids.reshape(batch, 2, HALFSEQ)
    pos = lax.slice(position_table, (1, 0), (1 + seq, d))
    out = _make_emb_kernel(batch)(ids3d, pos, charactor_embeddings)
    return out.reshape(batch, seq, d)
